# Initial kernel scaffold; baseline (speedup 1.0000x reference)
import jax
import jax.numpy as jnp
from jax.experimental import pallas as pl


def kernel(x, table):
    raise NotImplementedError("write your pallas kernel here")


# speedup vs baseline: 1.0153x; 1.0153x over previous
import functools
import math

import jax
import jax.numpy as jnp
from jax import lax
from jax.experimental import pallas as pl
from jax.experimental.pallas import tpu as pltpu
from jax.experimental.pallas import tpu_sc as plsc

D_MODEL = 1024
SCALE = math.sqrt(D_MODEL)
LANES = 16
VECS_PER_ROW = D_MODEL // LANES


@functools.lru_cache(maxsize=None)
def _build_sc_embed(B, num_cores, num_subcores, C):
    NW = num_cores * num_subcores
    b_per_w = B // NW
    n_chunks = b_per_w // C
    mesh = plsc.VectorSubcoreMesh(core_axis_name="c", subcore_axis_name="s")

    @functools.partial(
        pl.kernel,
        mesh=mesh,
        out_type=jax.ShapeDtypeStruct((B, D_MODEL), jnp.float32),
        scratch_types=[
            pltpu.VMEM((b_per_w,), jnp.int32),
            pltpu.VMEM((C, D_MODEL), jnp.float32),
            pltpu.SemaphoreType.DMA,
        ],
    )
    def sc_embed(idx_hbm, table_hbm, out_hbm, idx_v, rows_v, gsem):
        wid = lax.axis_index("s") * num_cores + lax.axis_index("c")
        base = wid * b_per_w
        pltpu.sync_copy(idx_hbm.at[pl.ds(base, b_per_w)], idx_v)

        for g in range(n_chunks):
            pltpu.async_copy(
                table_hbm.at[idx_v.at[pl.ds(g * C, C)]], rows_v, gsem
            ).wait()

            def row_body(r, carry):
                for c in range(VECS_PER_ROW):
                    s = pl.ds(c * LANES, LANES)
                    rows_v[r, s] = rows_v[r, s] * SCALE
                return carry

            lax.fori_loop(0, C, row_body, 0)

            pltpu.sync_copy(rows_v, out_hbm.at[pl.ds(base + g * C, C)])

    return sc_embed


def kernel(x, table):
    B = x.shape[0] * x.shape[1]
    idx = x.reshape(B).astype(jnp.int32)
    out = _build_sc_embed(B, 2, 16, 64)(idx, table)
    return out.reshape(x.shape[0], x.shape[1], D_MODEL)


# speedup vs baseline: 1.4754x; 1.4532x over previous
import functools
import math

import jax
import jax.numpy as jnp
from jax import lax
from jax.experimental import pallas as pl
from jax.experimental.pallas import tpu as pltpu
from jax.experimental.pallas import tpu_sc as plsc

D_MODEL = 1024
SCALE = math.sqrt(D_MODEL)
LANES = 16
VECS_PER_ROW = D_MODEL // LANES


@functools.lru_cache(maxsize=None)
def _build_sc_embed(B, num_cores, num_subcores, C, NBUF):
    NW = num_cores * num_subcores
    b_per_w = B // NW
    n_chunks = b_per_w // C
    mesh = plsc.VectorSubcoreMesh(core_axis_name="c", subcore_axis_name="s")

    @functools.partial(
        pl.kernel,
        mesh=mesh,
        out_type=jax.ShapeDtypeStruct((B, D_MODEL), jnp.float32),
        scratch_types=[
            pltpu.VMEM((b_per_w,), jnp.int32),
            *[pltpu.VMEM((C, D_MODEL), jnp.float32) for _ in range(NBUF)],
            *[pltpu.SemaphoreType.DMA for _ in range(2 * NBUF)],
        ],
    )
    def sc_embed(idx_hbm, table_hbm, out_hbm, idx_v, *bufs_and_sems):
        rows = bufs_and_sems[:NBUF]
        gsem = bufs_and_sems[NBUF : 2 * NBUF]
        ssem = bufs_and_sems[2 * NBUF : 3 * NBUF]

        wid = lax.axis_index("s") * num_cores + lax.axis_index("c")
        base = wid * b_per_w
        pltpu.sync_copy(idx_hbm.at[pl.ds(base, b_per_w)], idx_v)

        def start_gather(g):
            b = g % NBUF
            return pltpu.async_copy(
                table_hbm.at[idx_v.at[pl.ds(g * C, C)]], rows[b], gsem[b]
            )

        def start_scatter(g):
            b = g % NBUF
            return pltpu.async_copy(
                rows[b], out_hbm.at[pl.ds(base + g * C, C)], ssem[b]
            )

        gathers = [None] * n_chunks
        scatters = [None] * n_chunks
        drained = [False] * n_chunks
        for g in range(min(NBUF - 1, n_chunks)):
            gathers[g] = start_gather(g)

        for g in range(n_chunks):
            b = g % NBUF
            gathers[g].wait()

            def row_body(r, carry, rv=rows[b]):
                for c in range(VECS_PER_ROW):
                    s = pl.ds(c * LANES, LANES)
                    rv[r, s] = rv[r, s] * SCALE
                return carry

            lax.fori_loop(0, C, row_body, 0)

            scatters[g] = start_scatter(g)

            nxt = g + NBUF - 1
            if nxt < n_chunks:
                pg = nxt - NBUF
                if pg >= 0:
                    scatters[pg].wait()
                    drained[pg] = True
                gathers[nxt] = start_gather(nxt)

        for g in range(n_chunks):
            if scatters[g] is not None and not drained[g]:
                scatters[g].wait()

    return sc_embed


def kernel(x, table):
    B = x.shape[0] * x.shape[1]
    idx = x.reshape(B).astype(jnp.int32)
    out = _build_sc_embed(B, 2, 16, 32, 3)(idx, table)
    return out.reshape(x.shape[0], x.shape[1], D_MODEL)
